# baseline (speedup 1.0000x reference)
import functools

import numpy as np
import jax
import jax.numpy as jnp
from jax import lax
from jax.experimental import pallas as pl
from jax.experimental.pallas import tpu as pltpu
from jax.experimental.pallas import tpu_sc as plsc

_VOCAB = 1000000
_EMBED = 64
_SEQ = 200
_BATCH = 4096
_N = _BATCH * _SEQ

_NW = 32
_PW = _N // _NW
_G = 100
_K = 8
_C = _G * _K
_NCHUNK = _PW // _C
_LANES = 16
_DV = _EMBED // _LANES
_SPC = _C // _SEQ


def _make_pe():
    pos = np.arange(_SEQ, dtype=np.float32)[:, None]
    div = np.exp(np.arange(0, _EMBED, 2, dtype=np.float32)
                 * -(np.log(10000.0) / _EMBED))
    pe = np.zeros((_SEQ, _EMBED), np.float32)
    pe[:, 0::2] = np.sin(pos * div)
    pe[:, 1::2] = np.cos(pos * div)
    return pe


_PE = _make_pe()


def _sc_embed(seq2d, table, pe):
    mesh = plsc.VectorSubcoreMesh(core_axis_name="c", subcore_axis_name="s")

    @functools.partial(
        pl.kernel,
        mesh=mesh,
        out_type=jax.ShapeDtypeStruct((_N, _EMBED), jnp.float32),
        scratch_types=[
            pltpu.VMEM((_K, _G), jnp.int32),
            pltpu.VMEM((_C, _EMBED), jnp.float32),
            pltpu.VMEM((_SEQ, _EMBED), jnp.float32),
            pltpu.SemaphoreType.DMA,
        ],
        compiler_params=pltpu.CompilerParams(use_tc_tiling_on_sc=False),
    )
    def k(seq_hbm, table_hbm, pe_hbm, out_hbm, idx_v, rows_v, pe_v, sem):
        wid = lax.axis_index("s") * 2 + lax.axis_index("c")
        base = wid * _PW
        pltpu.sync_copy(pe_hbm, pe_v)

        def chunk_body(i, carry):
            off = base + i * _C
            srow = pl.multiple_of(off // _G, 8)
            pltpu.sync_copy(seq_hbm.at[pl.ds(srow, _K)], idx_v)
            copies = [
                pltpu.async_copy(
                    table_hbm.at[idx_v.at[j]],
                    rows_v.at[pl.ds(j * _G, _G)],
                    sem,
                )
                for j in range(_K)
            ]
            for c in copies:
                c.wait()

            def s_body(s, carry2):
                for d in range(_DV):
                    pv = pe_v[s, pl.ds(d * _LANES, _LANES)]
                    for j in range(_SPC):
                        r = j * _SEQ + s
                        sl = pl.ds(d * _LANES, _LANES)
                        rows_v[r, sl] = rows_v[r, sl] + pv
                return carry2

            lax.fori_loop(0, _SEQ, s_body, 0)
            pltpu.sync_copy(rows_v, out_hbm.at[pl.ds(off, _C)])
            return carry

        lax.fori_loop(0, _NCHUNK, chunk_body, 0)

    return k(seq2d, table, pe)


@jax.jit
def kernel(sequence, table):
    seq2d = sequence.astype(jnp.int32).reshape(_N // _G, _G)
    pe = jnp.asarray(_PE)
    out = _sc_embed(seq2d, table, pe)
    return out.reshape(_BATCH, _SEQ, _EMBED)
